# baseline (speedup 1.0000x reference)
import jax
import jax.numpy as jnp
from jax import lax
from jax.experimental import pallas as pl
from jax.experimental.pallas import tpu as pltpu
from jax.experimental.pallas import tpu_sc as plsc

_N_TOK = 16384
_N_SRC = 1024
_TAU = 1.0

_SC_ROWS = 3072
_TC_ROWS = _N_TOK - _SC_ROWS

_L = 16
_NC, _NS = 2, 16
_NW = _NC * _NS
_ROWS_W = _SC_ROWS // _NW
_CHUNK = 16
_NCHUNK = _ROWS_W // _CHUNK
_CVECS = _N_SRC // _L
_G = 16
_NBUF = 4

_RB = 1024
_NB = _TC_ROWS // _RB


def _sc_body(src_hbm, ew_hbm, lg_hbm, outx_hbm, outy_hbm,
             lg_v, ew_v, w_v, buf, out_v, tr_v,
             sem0, sem1, sem2, sem3, sem_p0, sem_p1):
    wid = lax.axis_index("c") * _NS + lax.axis_index("s")
    base = _TC_ROWS + wid * _ROWS_W
    lane = lax.iota(jnp.int32, _L)

    sems = (sem0, sem1, sem2, sem3)

    def issue(i, b):
        pltpu.async_copy(src_hbm.at[pl.ds(base + i * _CHUNK, _CHUNK)],
                         buf.at[b], sems[b])

    def wait(b):
        pltpu.make_async_copy(src_hbm.at[pl.ds(0, _CHUNK)],
                              buf.at[b], sems[b]).wait()

    for p in range(min(_NBUF, _NCHUNK)):
        issue(p, p)
    lg_cp = pltpu.async_copy(lg_hbm, lg_v, sem_p0)
    ew_cp = pltpu.async_copy(ew_hbm, ew_v, sem_p1)

    def bcast_gather(col):
        return plsc.load_gather(
            tr_v, [jnp.zeros((_L,), jnp.int32),
                   jnp.full((_L,), col, jnp.int32)])

    def col_of(col):
        return plsc.load_gather(tr_v, [lane, jnp.full((_L,), col, jnp.int32)])

    lg_cp.wait()
    ew_cp.wait()
    for r in range(2):
        m = lg_v[r, pl.ds(0, _L)] * (1.0 / _TAU)
        for c in range(1, _CVECS):
            m = jnp.maximum(m, lg_v[r, pl.ds(c * _L, _L)] * (1.0 / _TAU))
        tr_v[0, :] = m
        mx = bcast_gather(0)
        for c in range(1, _L):
            mx = jnp.maximum(mx, bcast_gather(c))
        s = jnp.zeros((_L,), jnp.float32)
        for c in range(_CVECS):
            e = jnp.exp(lg_v[r, pl.ds(c * _L, _L)] * (1.0 / _TAU) - mx)
            w_v[r, pl.ds(c * _L, _L)] = e
            s = s + e
        tr_v[0, :] = s
        tot = bcast_gather(0)
        for c in range(1, _L):
            tot = tot + bcast_gather(c)
        inv = 1.0 / tot
        for c in range(_CVECS):
            sl = pl.ds(c * _L, _L)
            w_v[r, sl] = w_v[r, sl] * inv * ew_v[sl]

    def group(b, g_row):
        def col_body(c, carry):
            a0, a1 = carry
            off = c * _L
            w0 = w_v[0, pl.ds(off, _L)]
            w1 = w_v[1, pl.ds(off, _L)]
            n0, n1 = [], []
            for j in range(_G):
                v = buf[b, g_row + j, pl.ds(off, _L)]
                n0.append(a0[j] + v * w0)
                n1.append(a1[j] + v * w1)
            return tuple(n0), tuple(n1)

        z = tuple(jnp.zeros((_L,), jnp.float32) for _ in range(_G))
        a0, a1 = lax.fori_loop(0, _CVECS, col_body, (z, z))
        outs = []
        for a in (a0, a1):
            for j in range(_G):
                tr_v[j, :] = a[j]
            t = col_of(0)
            for c in range(1, _L):
                t = t + col_of(c)
            outs.append(t)
        return outs[0], outs[1]

    for i in range(_NCHUNK):
        b = i % _NBUF
        wait(b)
        for g in range(_CHUNK // _G):
            o0, o1 = group(b, g * _G)
            loc = i * _CHUNK + g * _G
            out_v[0, pl.ds(loc, _G)] = o0
            out_v[1, pl.ds(loc, _G)] = o1
        if i + _NBUF < _NCHUNK:
            issue(i + _NBUF, b)

    pltpu.sync_copy(out_v.at[0], outx_hbm.at[pl.ds(wid * _ROWS_W, _ROWS_W)])
    pltpu.sync_copy(out_v.at[1], outy_hbm.at[pl.ds(wid * _ROWS_W, _ROWS_W)])


_sc_router = pl.kernel(
    _sc_body,
    out_type=(
        jax.ShapeDtypeStruct((_SC_ROWS,), jnp.float32),
        jax.ShapeDtypeStruct((_SC_ROWS,), jnp.float32),
    ),
    mesh=plsc.VectorSubcoreMesh(core_axis_name="c", subcore_axis_name="s"),
    compiler_params=pltpu.CompilerParams(needs_layout_passes=False),
    scratch_types=[
        pltpu.VMEM((2, _N_SRC), jnp.float32),
        pltpu.VMEM((_N_SRC,), jnp.float32),
        pltpu.VMEM((2, _N_SRC), jnp.float32),
        pltpu.VMEM((_NBUF, _CHUNK, _N_SRC), jnp.float32),
        pltpu.VMEM((2, _ROWS_W), jnp.float32),
        pltpu.VMEM((_G, _L), jnp.float32),
        pltpu.SemaphoreType.DMA,
        pltpu.SemaphoreType.DMA,
        pltpu.SemaphoreType.DMA,
        pltpu.SemaphoreType.DMA,
        pltpu.SemaphoreType.DMA,
        pltpu.SemaphoreType.DMA,
    ],
)


def _tc_body(ew_ref, lg_ref, src_ref, ox_ref, oy_ref, wpad_ref):
    @pl.when(pl.program_id(0) == 0)
    def _():
        lg = lg_ref[...] * (1.0 / _TAU)
        m = jnp.max(lg, axis=1, keepdims=True)
        e = jnp.exp(lg - m)
        p = e / jnp.sum(e, axis=1, keepdims=True)
        w2 = p * ew_ref[...][None, :]
        r = lax.broadcasted_iota(jnp.int32, (_N_SRC, 8), 1)
        wpad_ref[...] = (jnp.where(r == 0, w2[0, :][:, None], 0.0)
                         + jnp.where(r == 1, w2[1, :][:, None], 0.0))

    res = lax.dot_general(src_ref[...], wpad_ref[...],
                          (((1,), (0,)), ((), ())),
                          preferred_element_type=jnp.float32)
    ox_ref[...] = res[:, 0:1]
    oy_ref[...] = res[:, 1:2]


_tc_matvec = pl.pallas_call(
    _tc_body,
    grid=(_NB,),
    in_specs=[
        pl.BlockSpec((_N_SRC,), lambda i: (0,)),
        pl.BlockSpec((2, _N_SRC), lambda i: (0, 0)),
        pl.BlockSpec((_RB, _N_SRC), lambda i: (i, 0)),
    ],
    out_specs=[
        pl.BlockSpec((_RB, 1), lambda i: (i, 0)),
        pl.BlockSpec((_RB, 1), lambda i: (i, 0)),
    ],
    out_shape=[
        jax.ShapeDtypeStruct((_N_TOK, 1), jnp.float32),
        jax.ShapeDtypeStruct((_N_TOK, 1), jnp.float32),
    ],
    scratch_shapes=[pltpu.VMEM((_N_SRC, 8), jnp.float32)],
)


@jax.jit
def kernel(sources, edge_weights, route_logits):
    xs, ys = _sc_router(sources, edge_weights, route_logits)
    xt, yt = _tc_matvec(edge_weights, route_logits, sources)
    x = lax.dynamic_update_slice(xt.reshape(_N_TOK), xs, (_TC_ROWS,))
    y = lax.dynamic_update_slice(yt.reshape(_N_TOK), ys, (_TC_ROWS,))
    return (x, y)


# speedup vs baseline: 1.2694x; 1.2694x over previous
import jax
import jax.numpy as jnp
from jax import lax
from jax.experimental import pallas as pl
from jax.experimental.pallas import tpu as pltpu
from jax.experimental.pallas import tpu_sc as plsc

_N_TOK = 16384
_N_SRC = 1024
_TAU = 1.0

_SC_ROWS = 3072
_TC_ROWS = _N_TOK - _SC_ROWS

_L = 16
_NC, _NS = 2, 16
_NW = _NC * _NS
_ROWS_W = _SC_ROWS // _NW
_CHUNK = 16
_NCHUNK = _ROWS_W // _CHUNK
_CVECS = _N_SRC // _L
_G = 16
_NBUF = 4

_RB = 1024
_NB = _TC_ROWS // _RB


def _sc_body(src_hbm, ew_hbm, lg_hbm, outx_hbm, outy_hbm,
             lg_v, ew_v, w_v, buf, out_v, tr_v,
             sem0, sem1, sem2, sem3, sem_p0, sem_p1):
    wid = lax.axis_index("c") * _NS + lax.axis_index("s")
    base = _TC_ROWS + wid * _ROWS_W
    lane = lax.iota(jnp.int32, _L)

    sems = (sem0, sem1, sem2, sem3)

    def issue(i, b):
        pltpu.async_copy(src_hbm.at[pl.ds(base + i * _CHUNK, _CHUNK)],
                         buf.at[b], sems[b])

    def wait(b):
        pltpu.make_async_copy(src_hbm.at[pl.ds(0, _CHUNK)],
                              buf.at[b], sems[b]).wait()

    for p in range(min(_NBUF, _NCHUNK)):
        issue(p, p)
    lg_cp = pltpu.async_copy(lg_hbm, lg_v, sem_p0)
    ew_cp = pltpu.async_copy(ew_hbm, ew_v, sem_p1)

    def bcast_gather(col):
        return plsc.load_gather(
            tr_v, [jnp.zeros((_L,), jnp.int32),
                   jnp.full((_L,), col, jnp.int32)])

    def col_of(col):
        return plsc.load_gather(tr_v, [lane, jnp.full((_L,), col, jnp.int32)])

    lg_cp.wait()
    ew_cp.wait()
    for r in range(2):
        m = lg_v[r, pl.ds(0, _L)] * (1.0 / _TAU)
        for c in range(1, _CVECS):
            m = jnp.maximum(m, lg_v[r, pl.ds(c * _L, _L)] * (1.0 / _TAU))
        tr_v[0, :] = m
        mx = bcast_gather(0)
        for c in range(1, _L):
            mx = jnp.maximum(mx, bcast_gather(c))
        s = jnp.zeros((_L,), jnp.float32)
        for c in range(_CVECS):
            e = jnp.exp(lg_v[r, pl.ds(c * _L, _L)] * (1.0 / _TAU) - mx)
            w_v[r, pl.ds(c * _L, _L)] = e
            s = s + e
        tr_v[0, :] = s
        tot = bcast_gather(0)
        for c in range(1, _L):
            tot = tot + bcast_gather(c)
        inv = 1.0 / tot
        for c in range(_CVECS):
            sl = pl.ds(c * _L, _L)
            w_v[r, sl] = w_v[r, sl] * inv * ew_v[sl]

    def group(b, g_row):
        def col_body(c, carry):
            a0, a1 = carry
            off = c * _L
            w0 = w_v[0, pl.ds(off, _L)]
            w1 = w_v[1, pl.ds(off, _L)]
            n0, n1 = [], []
            for j in range(_G):
                v = buf[b, g_row + j, pl.ds(off, _L)]
                n0.append(a0[j] + v * w0)
                n1.append(a1[j] + v * w1)
            return tuple(n0), tuple(n1)

        z = tuple(jnp.zeros((_L,), jnp.float32) for _ in range(_G))
        a0, a1 = lax.fori_loop(0, _CVECS, col_body, (z, z))
        outs = []
        for a in (a0, a1):
            for j in range(_G):
                tr_v[j, :] = a[j]
            t = col_of(0)
            for c in range(1, _L):
                t = t + col_of(c)
            outs.append(t)
        return outs[0], outs[1]

    for i in range(_NCHUNK):
        b = i % _NBUF
        wait(b)
        for g in range(_CHUNK // _G):
            o0, o1 = group(b, g * _G)
            loc = i * _CHUNK + g * _G
            out_v[0, pl.ds(loc, _G)] = o0
            out_v[1, pl.ds(loc, _G)] = o1
        if i + _NBUF < _NCHUNK:
            issue(i + _NBUF, b)

    pltpu.sync_copy(out_v.at[0], outx_hbm.at[pl.ds(wid * _ROWS_W, _ROWS_W)])
    pltpu.sync_copy(out_v.at[1], outy_hbm.at[pl.ds(wid * _ROWS_W, _ROWS_W)])


_sc_router = pl.kernel(
    _sc_body,
    out_type=(
        jax.ShapeDtypeStruct((_SC_ROWS,), jnp.float32),
        jax.ShapeDtypeStruct((_SC_ROWS,), jnp.float32),
    ),
    mesh=plsc.VectorSubcoreMesh(core_axis_name="c", subcore_axis_name="s"),
    compiler_params=pltpu.CompilerParams(needs_layout_passes=False),
    scratch_types=[
        pltpu.VMEM((2, _N_SRC), jnp.float32),
        pltpu.VMEM((_N_SRC,), jnp.float32),
        pltpu.VMEM((2, _N_SRC), jnp.float32),
        pltpu.VMEM((_NBUF, _CHUNK, _N_SRC), jnp.float32),
        pltpu.VMEM((2, _ROWS_W), jnp.float32),
        pltpu.VMEM((_G, _L), jnp.float32),
        pltpu.SemaphoreType.DMA,
        pltpu.SemaphoreType.DMA,
        pltpu.SemaphoreType.DMA,
        pltpu.SemaphoreType.DMA,
        pltpu.SemaphoreType.DMA,
        pltpu.SemaphoreType.DMA,
    ],
)


def _tc_body(ew_ref, lg_ref, src_ref, o_ref, wpad_ref):
    @pl.when(pl.program_id(0) == 0)
    def _():
        lg = lg_ref[...] * (1.0 / _TAU)
        m = jnp.max(lg, axis=1, keepdims=True)
        e = jnp.exp(lg - m)
        p = e / jnp.sum(e, axis=1, keepdims=True)
        w2 = p * ew_ref[...][None, :]
        r = lax.broadcasted_iota(jnp.int32, (_N_SRC, 8), 1)
        wpad_ref[...] = (jnp.where(r == 0, w2[0, :][:, None], 0.0)
                         + jnp.where(r == 1, w2[1, :][:, None], 0.0))

    res = lax.dot_general(src_ref[...], wpad_ref[...],
                          (((1,), (0,)), ((), ())),
                          preferred_element_type=jnp.float32)
    o_ref[...] = res.T


_tc_matvec = pl.pallas_call(
    _tc_body,
    grid=(_NB,),
    in_specs=[
        pl.BlockSpec((_N_SRC,), lambda i: (0,)),
        pl.BlockSpec((2, _N_SRC), lambda i: (0, 0)),
        pl.BlockSpec((_RB, _N_SRC), lambda i: (i, 0)),
    ],
    out_specs=[
        pl.BlockSpec((8, _RB), lambda i: (0, i)),
    ],
    out_shape=[
        jax.ShapeDtypeStruct((8, _N_TOK), jnp.float32),
    ],
    scratch_shapes=[pltpu.VMEM((_N_SRC, 8), jnp.float32)],
)


@jax.jit
def kernel(sources, edge_weights, route_logits):
    xs, ys = _sc_router(sources, edge_weights, route_logits)
    ot = _tc_matvec(edge_weights, route_logits, sources)[0]
    x = lax.dynamic_update_slice(ot[0], xs, (_TC_ROWS,))
    y = lax.dynamic_update_slice(ot[1], ys, (_TC_ROWS,))
    return (x, y)


# speedup vs baseline: 1.2788x; 1.0074x over previous
import jax
import jax.numpy as jnp
from jax import lax
from jax.experimental import pallas as pl
from jax.experimental.pallas import tpu as pltpu
from jax.experimental.pallas import tpu_sc as plsc

_N_TOK = 16384
_N_SRC = 1024
_TAU = 1.0

_SC_ROWS = 4096
_TC_ROWS = _N_TOK - _SC_ROWS

_L = 16
_NC, _NS = 2, 16
_NW = _NC * _NS
_ROWS_W = _SC_ROWS // _NW
_CHUNK = 16
_NCHUNK = _ROWS_W // _CHUNK
_CVECS = _N_SRC // _L
_G = 16
_NBUF = 4

_RB = 2048
_NB = _TC_ROWS // _RB


def _sc_body(src_hbm, ew_hbm, lg_hbm, outx_hbm, outy_hbm,
             lg_v, ew_v, w_v, buf, out_v, tr_v,
             sem0, sem1, sem2, sem3, sem_p0, sem_p1):
    wid = lax.axis_index("c") * _NS + lax.axis_index("s")
    base = _TC_ROWS + wid * _ROWS_W
    lane = lax.iota(jnp.int32, _L)

    sems = (sem0, sem1, sem2, sem3)

    def issue(i, b):
        pltpu.async_copy(src_hbm.at[pl.ds(base + i * _CHUNK, _CHUNK)],
                         buf.at[b], sems[b])

    def wait(b):
        pltpu.make_async_copy(src_hbm.at[pl.ds(0, _CHUNK)],
                              buf.at[b], sems[b]).wait()

    for p in range(min(_NBUF, _NCHUNK)):
        issue(p, p)
    lg_cp = pltpu.async_copy(lg_hbm, lg_v, sem_p0)
    ew_cp = pltpu.async_copy(ew_hbm, ew_v, sem_p1)

    def bcast_gather(col):
        return plsc.load_gather(
            tr_v, [jnp.zeros((_L,), jnp.int32),
                   jnp.full((_L,), col, jnp.int32)])

    def col_of(col):
        return plsc.load_gather(tr_v, [lane, jnp.full((_L,), col, jnp.int32)])

    lg_cp.wait()
    ew_cp.wait()
    for r in range(2):
        m = lg_v[r, pl.ds(0, _L)] * (1.0 / _TAU)
        for c in range(1, _CVECS):
            m = jnp.maximum(m, lg_v[r, pl.ds(c * _L, _L)] * (1.0 / _TAU))
        tr_v[0, :] = m
        mx = bcast_gather(0)
        for c in range(1, _L):
            mx = jnp.maximum(mx, bcast_gather(c))
        s = jnp.zeros((_L,), jnp.float32)
        for c in range(_CVECS):
            e = jnp.exp(lg_v[r, pl.ds(c * _L, _L)] * (1.0 / _TAU) - mx)
            w_v[r, pl.ds(c * _L, _L)] = e
            s = s + e
        tr_v[0, :] = s
        tot = bcast_gather(0)
        for c in range(1, _L):
            tot = tot + bcast_gather(c)
        inv = 1.0 / tot
        for c in range(_CVECS):
            sl = pl.ds(c * _L, _L)
            w_v[r, sl] = w_v[r, sl] * inv * ew_v[sl]

    def group(b, g_row):
        def col_body(c, carry):
            a0, a1 = carry
            off = c * _L
            w0 = w_v[0, pl.ds(off, _L)]
            w1 = w_v[1, pl.ds(off, _L)]
            n0, n1 = [], []
            for j in range(_G):
                v = buf[b, g_row + j, pl.ds(off, _L)]
                n0.append(a0[j] + v * w0)
                n1.append(a1[j] + v * w1)
            return tuple(n0), tuple(n1)

        z = tuple(jnp.zeros((_L,), jnp.float32) for _ in range(_G))
        a0, a1 = lax.fori_loop(0, _CVECS, col_body, (z, z))
        outs = []
        for a in (a0, a1):
            for j in range(_G):
                tr_v[j, :] = a[j]
            t = col_of(0)
            for c in range(1, _L):
                t = t + col_of(c)
            outs.append(t)
        return outs[0], outs[1]

    for i in range(_NCHUNK):
        b = i % _NBUF
        wait(b)
        for g in range(_CHUNK // _G):
            o0, o1 = group(b, g * _G)
            loc = i * _CHUNK + g * _G
            out_v[0, pl.ds(loc, _G)] = o0
            out_v[1, pl.ds(loc, _G)] = o1
        if i + _NBUF < _NCHUNK:
            issue(i + _NBUF, b)

    pltpu.sync_copy(out_v.at[0], outx_hbm.at[pl.ds(wid * _ROWS_W, _ROWS_W)])
    pltpu.sync_copy(out_v.at[1], outy_hbm.at[pl.ds(wid * _ROWS_W, _ROWS_W)])


_sc_router = pl.kernel(
    _sc_body,
    out_type=(
        jax.ShapeDtypeStruct((_SC_ROWS,), jnp.float32),
        jax.ShapeDtypeStruct((_SC_ROWS,), jnp.float32),
    ),
    mesh=plsc.VectorSubcoreMesh(core_axis_name="c", subcore_axis_name="s"),
    compiler_params=pltpu.CompilerParams(needs_layout_passes=False),
    scratch_types=[
        pltpu.VMEM((2, _N_SRC), jnp.float32),
        pltpu.VMEM((_N_SRC,), jnp.float32),
        pltpu.VMEM((2, _N_SRC), jnp.float32),
        pltpu.VMEM((_NBUF, _CHUNK, _N_SRC), jnp.float32),
        pltpu.VMEM((2, _ROWS_W), jnp.float32),
        pltpu.VMEM((_G, _L), jnp.float32),
        pltpu.SemaphoreType.DMA,
        pltpu.SemaphoreType.DMA,
        pltpu.SemaphoreType.DMA,
        pltpu.SemaphoreType.DMA,
        pltpu.SemaphoreType.DMA,
        pltpu.SemaphoreType.DMA,
    ],
)


def _tc_body(ew_ref, lg_ref, src_ref, ox_ref, oy_ref, wpad_ref):
    @pl.when(pl.program_id(0) == 0)
    def _():
        lg = lg_ref[...] * (1.0 / _TAU)
        m = jnp.max(lg, axis=1, keepdims=True)
        e = jnp.exp(lg - m)
        p = e / jnp.sum(e, axis=1, keepdims=True)
        w2 = p * ew_ref[...][None, :]
        r = lax.broadcasted_iota(jnp.int32, (8, _N_SRC), 0)
        wpad_ref[...] = (jnp.where(r == 0, w2[0, :][None, :], 0.0)
                         + jnp.where(r == 1, w2[1, :][None, :], 0.0))

    res = lax.dot_general(wpad_ref[...], src_ref[...],
                          (((1,), (1,)), ((), ())),
                          preferred_element_type=jnp.float32)
    ox_ref[...] = res[0, :][None, None, :]
    oy_ref[...] = res[1, :][None, None, :]


_tc_matvec = pl.pallas_call(
    _tc_body,
    grid=(_NB,),
    in_specs=[
        pl.BlockSpec((_N_SRC,), lambda i: (0,)),
        pl.BlockSpec((2, _N_SRC), lambda i: (0, 0)),
        pl.BlockSpec((_RB, _N_SRC), lambda i: (i, 0)),
    ],
    out_specs=[
        pl.BlockSpec((1, 1, _RB), lambda i: (i, 0, 0)),
        pl.BlockSpec((1, 1, _RB), lambda i: (i, 0, 0)),
    ],
    out_shape=[
        jax.ShapeDtypeStruct((_N_TOK // _RB, 1, _RB), jnp.float32),
        jax.ShapeDtypeStruct((_N_TOK // _RB, 1, _RB), jnp.float32),
    ],
    scratch_shapes=[pltpu.VMEM((8, _N_SRC), jnp.float32)],
)


@jax.jit
def kernel(sources, edge_weights, route_logits):
    xs, ys = _sc_router(sources, edge_weights, route_logits)
    xt, yt = _tc_matvec(edge_weights, route_logits, sources)
    x = lax.dynamic_update_slice(xt.reshape(_N_TOK), xs, (_TC_ROWS,))
    y = lax.dynamic_update_slice(yt.reshape(_N_TOK), ys, (_TC_ROWS,))
    return (x, y)


# speedup vs baseline: 1.3958x; 1.0914x over previous
import jax
import jax.numpy as jnp
from jax import lax
from jax.experimental import pallas as pl
from jax.experimental.pallas import tpu as pltpu
from jax.experimental.pallas import tpu_sc as plsc

_N_TOK = 16384
_N_SRC = 1024
_TAU = 1.0

_SC_ROWS = 4096
_TC_ROWS = _N_TOK - _SC_ROWS

_L = 16
_NC, _NS = 2, 16
_NW = _NC * _NS
_ROWS_W = _SC_ROWS // _NW
_CHUNK = 16
_NCHUNK = _ROWS_W // _CHUNK
_CVECS = _N_SRC // _L
_G = 16
_NBUF = 4

_RB = 2048
_NB = _TC_ROWS // _RB


def _softmax_body(ew_ref, lg_ref, w_ref):
    lg = lg_ref[...] * (1.0 / _TAU)
    m = jnp.max(lg, axis=1, keepdims=True)
    e = jnp.exp(lg - m)
    p = e / jnp.sum(e, axis=1, keepdims=True)
    w_ref[...] = p * ew_ref[...][None, :]


_softmax_w = pl.pallas_call(
    _softmax_body,
    out_shape=jax.ShapeDtypeStruct((2, _N_SRC), jnp.float32),
)


def _sc_body(src_hbm, w_hbm, outx_hbm, outy_hbm,
             w_v, buf, out_v, tr_v, sem0, sem1, sem2, sem3, sem_w):
    wid = lax.axis_index("c") * _NS + lax.axis_index("s")
    base = _TC_ROWS + wid * _ROWS_W
    lane = lax.iota(jnp.int32, _L)

    sems = (sem0, sem1, sem2, sem3)

    def issue(i, b):
        pltpu.async_copy(src_hbm.at[pl.ds(base + i * _CHUNK, _CHUNK)],
                         buf.at[b], sems[b])

    def wait(b):
        pltpu.make_async_copy(src_hbm.at[pl.ds(0, _CHUNK)],
                              buf.at[b], sems[b]).wait()

    for p in range(min(_NBUF, _NCHUNK)):
        issue(p, p)
    w_cp = pltpu.async_copy(w_hbm, w_v, sem_w)
    w_cp.wait()

    def col_of(col):
        return plsc.load_gather(tr_v, [lane, jnp.full((_L,), col, jnp.int32)])

    def group(b, g_row):
        def col_body(c, carry):
            a0, a1 = carry
            off = c * _L
            w0 = w_v[0, pl.ds(off, _L)]
            w1 = w_v[1, pl.ds(off, _L)]
            n0, n1 = [], []
            for j in range(_G):
                v = buf[b, g_row + j, pl.ds(off, _L)]
                n0.append(a0[j] + v * w0)
                n1.append(a1[j] + v * w1)
            return tuple(n0), tuple(n1)

        z = tuple(jnp.zeros((_L,), jnp.float32) for _ in range(_G))
        a0, a1 = lax.fori_loop(0, _CVECS, col_body, (z, z))
        outs = []
        for a in (a0, a1):
            for j in range(_G):
                tr_v[j, :] = a[j]
            t = col_of(0)
            for c in range(1, _L):
                t = t + col_of(c)
            outs.append(t)
        return outs[0], outs[1]

    for i in range(_NCHUNK):
        b = i % _NBUF
        wait(b)
        for g in range(_CHUNK // _G):
            o0, o1 = group(b, g * _G)
            loc = i * _CHUNK + g * _G
            out_v[0, pl.ds(loc, _G)] = o0
            out_v[1, pl.ds(loc, _G)] = o1
        if i + _NBUF < _NCHUNK:
            issue(i + _NBUF, b)

    pltpu.sync_copy(out_v.at[0], outx_hbm.at[pl.ds(wid * _ROWS_W, _ROWS_W)])
    pltpu.sync_copy(out_v.at[1], outy_hbm.at[pl.ds(wid * _ROWS_W, _ROWS_W)])


_sc_router = pl.kernel(
    _sc_body,
    out_type=(
        jax.ShapeDtypeStruct((_SC_ROWS,), jnp.float32),
        jax.ShapeDtypeStruct((_SC_ROWS,), jnp.float32),
    ),
    mesh=plsc.VectorSubcoreMesh(core_axis_name="c", subcore_axis_name="s"),
    compiler_params=pltpu.CompilerParams(needs_layout_passes=False),
    scratch_types=[
        pltpu.VMEM((2, _N_SRC), jnp.float32),
        pltpu.VMEM((_NBUF, _CHUNK, _N_SRC), jnp.float32),
        pltpu.VMEM((2, _ROWS_W), jnp.float32),
        pltpu.VMEM((_G, _L), jnp.float32),
        pltpu.SemaphoreType.DMA,
        pltpu.SemaphoreType.DMA,
        pltpu.SemaphoreType.DMA,
        pltpu.SemaphoreType.DMA,
        pltpu.SemaphoreType.DMA,
    ],
)


def _tc_body(w_ref, src_ref, ox_ref, oy_ref, wpad_ref):
    @pl.when(pl.program_id(0) == 0)
    def _():
        r = lax.broadcasted_iota(jnp.int32, (8, _N_SRC), 0)
        wpad_ref[...] = (jnp.where(r == 0, w_ref[0, :][None, :], 0.0)
                         + jnp.where(r == 1, w_ref[1, :][None, :], 0.0))

    res = lax.dot_general(wpad_ref[...], src_ref[...],
                          (((1,), (1,)), ((), ())),
                          preferred_element_type=jnp.float32)
    ox_ref[...] = res[0, :][None, None, :]
    oy_ref[...] = res[1, :][None, None, :]


_tc_matvec = pl.pallas_call(
    _tc_body,
    grid=(_NB,),
    in_specs=[
        pl.BlockSpec((2, _N_SRC), lambda i: (0, 0)),
        pl.BlockSpec((_RB, _N_SRC), lambda i: (i, 0)),
    ],
    out_specs=[
        pl.BlockSpec((1, 1, _RB), lambda i: (i, 0, 0)),
        pl.BlockSpec((1, 1, _RB), lambda i: (i, 0, 0)),
    ],
    out_shape=[
        jax.ShapeDtypeStruct((_N_TOK // _RB, 1, _RB), jnp.float32),
        jax.ShapeDtypeStruct((_N_TOK // _RB, 1, _RB), jnp.float32),
    ],
    scratch_shapes=[pltpu.VMEM((8, _N_SRC), jnp.float32)],
)


@jax.jit
def kernel(sources, edge_weights, route_logits):
    w = _softmax_w(edge_weights, route_logits)
    xs, ys = _sc_router(sources, w)
    xt, yt = _tc_matvec(w, sources)
    x = lax.dynamic_update_slice(xt.reshape(_N_TOK), xs, (_TC_ROWS,))
    y = lax.dynamic_update_slice(yt.reshape(_N_TOK), ys, (_TC_ROWS,))
    return (x, y)


# speedup vs baseline: 2.4436x; 1.7507x over previous
import jax
import jax.numpy as jnp
from jax import lax
from jax.experimental import pallas as pl
from jax.experimental.pallas import tpu as pltpu
from jax.experimental.pallas import tpu_sc as plsc

_N_TOK = 16384
_N_SRC = 1024
_TAU = 1.0

_SC_ROWS = 4096
_TC_ROWS = _N_TOK - _SC_ROWS

_L = 16
_NC, _NS = 2, 16
_NW = _NC * _NS
_ROWS_W = _SC_ROWS // _NW
_CHUNK = 16
_NCHUNK = _ROWS_W // _CHUNK
_CVECS = _N_SRC // _L
_G = 16
_NBUF = 4

_RB = 2048
_NB = _TC_ROWS // _RB


def _softmax_body(ew_ref, lg_ref, w_ref):
    lg = lg_ref[...] * (1.0 / _TAU)
    m = jnp.max(lg, axis=1, keepdims=True)
    e = jnp.exp(lg - m)
    p = e / jnp.sum(e, axis=1, keepdims=True)
    w_ref[...] = p * ew_ref[...][None, :]


_softmax_w = pl.pallas_call(
    _softmax_body,
    out_shape=jax.ShapeDtypeStruct((2, _N_SRC), jnp.float32),
)


def _sc_body(src_hbm, w_hbm, outx_hbm, outy_hbm,
             w_v, buf, out_v, tr_v, sem0, sem1, sem2, sem3, sem_w):
    wid = lax.axis_index("c") * _NS + lax.axis_index("s")
    base = _TC_ROWS + wid * _ROWS_W
    lane = lax.iota(jnp.int32, _L)

    sems = (sem0, sem1, sem2, sem3)

    def issue(i, b):
        pltpu.async_copy(src_hbm.at[pl.ds(base + i * _CHUNK, _CHUNK)],
                         buf.at[b], sems[b])

    def wait(b):
        pltpu.make_async_copy(src_hbm.at[pl.ds(0, _CHUNK)],
                              buf.at[b], sems[b]).wait()

    for p in range(min(_NBUF, _NCHUNK)):
        issue(p, p)
    w_cp = pltpu.async_copy(w_hbm, w_v, sem_w)
    w_cp.wait()

    def col_of(col):
        return plsc.load_gather(tr_v, [lane, jnp.full((_L,), col, jnp.int32)])

    def group(b, g_row):
        def col_body(c, carry):
            a0, a1 = carry
            off = c * _L
            w0 = w_v[0, pl.ds(off, _L)]
            w1 = w_v[1, pl.ds(off, _L)]
            n0, n1 = [], []
            for j in range(_G):
                v = buf[b, g_row + j, pl.ds(off, _L)]
                n0.append(a0[j] + v * w0)
                n1.append(a1[j] + v * w1)
            return tuple(n0), tuple(n1)

        z = tuple(jnp.zeros((_L,), jnp.float32) for _ in range(_G))
        a0, a1 = lax.fori_loop(0, _CVECS, col_body, (z, z))
        outs = []
        for a in (a0, a1):
            for j in range(_G):
                tr_v[j, :] = a[j]
            t = col_of(0)
            for c in range(1, _L):
                t = t + col_of(c)
            outs.append(t)
        return outs[0], outs[1]

    for i in range(_NCHUNK):
        b = i % _NBUF
        wait(b)
        for g in range(_CHUNK // _G):
            o0, o1 = group(b, g * _G)
            loc = i * _CHUNK + g * _G
            out_v[0, pl.ds(loc, _G)] = o0
            out_v[1, pl.ds(loc, _G)] = o1
        if i + _NBUF < _NCHUNK:
            issue(i + _NBUF, b)

    pltpu.sync_copy(out_v.at[0], outx_hbm.at[pl.ds(wid * _ROWS_W, _ROWS_W)])
    pltpu.sync_copy(out_v.at[1], outy_hbm.at[pl.ds(wid * _ROWS_W, _ROWS_W)])


_sc_router = pl.kernel(
    _sc_body,
    out_type=(
        jax.ShapeDtypeStruct((_SC_ROWS,), jnp.float32),
        jax.ShapeDtypeStruct((_SC_ROWS,), jnp.float32),
    ),
    mesh=plsc.VectorSubcoreMesh(core_axis_name="c", subcore_axis_name="s"),
    compiler_params=pltpu.CompilerParams(needs_layout_passes=False),
    scratch_types=[
        pltpu.VMEM((2, _N_SRC), jnp.float32),
        pltpu.VMEM((_NBUF, _CHUNK, _N_SRC), jnp.float32),
        pltpu.VMEM((2, _ROWS_W), jnp.float32),
        pltpu.VMEM((_G, _L), jnp.float32),
        pltpu.SemaphoreType.DMA,
        pltpu.SemaphoreType.DMA,
        pltpu.SemaphoreType.DMA,
        pltpu.SemaphoreType.DMA,
        pltpu.SemaphoreType.DMA,
    ],
)


def _tc_body(w_ref, src_ref, ox_ref, oy_ref, wpad_ref):
    @pl.when(pl.program_id(0) == 0)
    def _():
        r = lax.broadcasted_iota(jnp.int32, (8, _N_SRC), 0)
        wpad_ref[...] = (jnp.where(r == 0, w_ref[0, :][None, :], 0.0)
                         + jnp.where(r == 1, w_ref[1, :][None, :], 0.0))

    res = lax.dot_general(wpad_ref[...], src_ref[...],
                          (((1,), (1,)), ((), ())),
                          preferred_element_type=jnp.float32)
    ox_ref[...] = res[0, :][None, None, :]
    oy_ref[...] = res[1, :][None, None, :]


_tc_matvec = pl.pallas_call(
    _tc_body,
    grid=(_NB,),
    in_specs=[
        pl.BlockSpec((2, _N_SRC), lambda i: (0, 0)),
        pl.BlockSpec((_RB, _N_SRC), lambda i: (i, 0)),
    ],
    out_specs=[
        pl.BlockSpec((1, 1, _RB), lambda i: (i, 0, 0)),
        pl.BlockSpec((1, 1, _RB), lambda i: (i, 0, 0)),
    ],
    out_shape=[
        jax.ShapeDtypeStruct((_N_TOK // _RB, 1, _RB), jnp.float32),
        jax.ShapeDtypeStruct((_N_TOK // _RB, 1, _RB), jnp.float32),
    ],
    scratch_shapes=[pltpu.VMEM((8, _N_SRC), jnp.float32)],
)


_tc_matvec_full = pl.pallas_call(
    _tc_body,
    grid=(_N_TOK // _RB,),
    in_specs=[
        pl.BlockSpec((2, _N_SRC), lambda i: (0, 0)),
        pl.BlockSpec((_RB, _N_SRC), lambda i: (i, 0)),
    ],
    out_specs=[
        pl.BlockSpec((1, 1, _RB), lambda i: (i, 0, 0)),
        pl.BlockSpec((1, 1, _RB), lambda i: (i, 0, 0)),
    ],
    out_shape=[
        jax.ShapeDtypeStruct((_N_TOK // _RB, 1, _RB), jnp.float32),
        jax.ShapeDtypeStruct((_N_TOK // _RB, 1, _RB), jnp.float32),
    ],
    scratch_shapes=[pltpu.VMEM((8, _N_SRC), jnp.float32)],
)


@jax.jit
def kernel(sources, edge_weights, route_logits):
    w = _softmax_w(edge_weights, route_logits)
    xt, yt = _tc_matvec_full(w, sources)
    return (xt.reshape(_N_TOK), yt.reshape(_N_TOK))
